# baseline (speedup 1.0000x reference)
import functools

import jax
import jax.numpy as jnp
from jax import lax
from jax.experimental import pallas as pl
from jax.experimental.pallas import tpu as pltpu
from jax.experimental.pallas import tpu_sc as plsc

VOCAB = 100000
D = 128
CTX = 200

_CH = 2
_CHN = CTX // _CH


def _sc_gather_sum(idx2, emb):
    mesh = plsc.VectorSubcoreMesh(core_axis_name="c", subcore_axis_name="s")

    @functools.partial(
        pl.kernel,
        out_type=jax.ShapeDtypeStruct((D,), jnp.float32),
        mesh=mesh,
        scratch_types=[
            pltpu.VMEM((_CH, _CHN), jnp.int32),
            pltpu.VMEM((_CH, _CHN, D), jnp.float32),
            pltpu.VMEM((D,), jnp.float32),
            pltpu.SemaphoreType.DMA,
        ],
    )
    def k(idx_hbm, emb_hbm, out_hbm, idx_v, rows_v, acc_v, sem):
        wid = lax.axis_index("s") * 2 + lax.axis_index("c")

        @pl.when(wid == 0)
        def _():
            pltpu.sync_copy(idx_hbm, idx_v)
            cps = [
                pltpu.async_copy(emb_hbm.at[idx_v.at[c]], rows_v.at[c], sem)
                for c in range(_CH)
            ]
            for cp in cps:
                cp.wait()

            def body(r, accs):
                new = []
                for d in range(D // 16):
                    a = accs[d]
                    for c in range(_CH):
                        a = a + rows_v[c, r, pl.ds(d * 16, 16)]
                    new.append(a)
                return tuple(new)

            accs = tuple(jnp.zeros((16,), jnp.float32) for _ in range(D // 16))
            accs = lax.fori_loop(0, _CHN, body, accs)
            for d in range(D // 16):
                acc_v[pl.ds(d * 16, 16)] = accs[d]
            pltpu.sync_copy(acc_v, out_hbm)

    return k(idx2, emb)


_BLK = 4096
_NB = (VOCAB + _BLK - 1) // _BLK
_LAST = VOCAB - (_NB - 1) * _BLK


def _k1_body(e_ref, w1_ref, b1_ref, w2_hbm, out_ref, logz_ref,
             h_ref, buf, last_buf, m_arr, s_arr, sem, sem_last):
    i = pl.program_id(0)

    @pl.when(i == 0)
    def _():
        h = jnp.dot(e_ref[...], w1_ref[...].T,
                    preferred_element_type=jnp.float32) + b1_ref[...]
        h_ref[...] = jnp.maximum(h, 0.0)
        pltpu.make_async_copy(
            w2_hbm.at[pl.ds(0, _BLK), :], buf.at[0], sem.at[0]).start()

    @pl.when(i + 1 < _NB - 1)
    def _():
        slot = lax.rem(i + 1, 2)
        pltpu.make_async_copy(
            w2_hbm.at[pl.ds((i + 1) * _BLK, _BLK), :], buf.at[slot],
            sem.at[slot]).start()

    @pl.when(i + 1 == _NB - 1)
    def _():
        pltpu.make_async_copy(
            w2_hbm.at[pl.ds((_NB - 1) * _BLK, _LAST), :], last_buf,
            sem_last).start()

    def stats(logits, bm):
        m_arr[i] = bm
        s_arr[i] = jnp.sum(jnp.exp(logits - bm))

    @pl.when(i < _NB - 1)
    def _():
        slot = lax.rem(i, 2)
        pltpu.make_async_copy(
            w2_hbm.at[pl.ds(i * _BLK, _BLK), :], buf.at[slot],
            sem.at[slot]).wait()
        logits = lax.dot_general(
            h_ref[...], buf[slot], (((1,), (1,)), ((), ())),
            preferred_element_type=jnp.float32)
        out_ref[...] = logits
        stats(logits, jnp.max(logits))

    @pl.when(i == _NB - 1)
    def _():
        pltpu.make_async_copy(
            w2_hbm.at[pl.ds((_NB - 1) * _BLK, _LAST), :], last_buf,
            sem_last).wait()
        logits = lax.dot_general(
            h_ref[...], last_buf[...], (((1,), (1,)), ((), ())),
            preferred_element_type=jnp.float32)
        out_ref[0:1, 0:_LAST] = logits
        stats(logits, jnp.max(logits))

        def comb(j, carry):
            m, s = carry
            mj = m_arr[j]
            mn = jnp.maximum(m, mj)
            return mn, s * jnp.exp(m - mn) + s_arr[j] * jnp.exp(mj - mn)

        m0, s0 = lax.fori_loop(1, _NB, comb, (m_arr[0], s_arr[0]))
        logz_ref[...] = jnp.full((1, D), m0 + jnp.log(s0), jnp.float32)


def _tc_logits(e, W1, b1, W2):
    return pl.pallas_call(
        _k1_body,
        grid=(_NB,),
        in_specs=[
            pl.BlockSpec((1, D), lambda i: (0, 0)),
            pl.BlockSpec((D, D), lambda i: (0, 0)),
            pl.BlockSpec((1, D), lambda i: (0, 0)),
            pl.BlockSpec(memory_space=pl.ANY),
        ],
        out_specs=[
            pl.BlockSpec((1, _BLK), lambda i: (0, i)),
            pl.BlockSpec((1, D), lambda i: (0, 0)),
        ],
        out_shape=[
            jax.ShapeDtypeStruct((1, VOCAB), jnp.float32),
            jax.ShapeDtypeStruct((1, D), jnp.float32),
        ],
        scratch_shapes=[
            pltpu.VMEM((1, D), jnp.float32),
            pltpu.VMEM((2, _BLK, D), jnp.float32),
            pltpu.VMEM((_LAST, D), jnp.float32),
            pltpu.SMEM((_NB,), jnp.float32),
            pltpu.SMEM((_NB,), jnp.float32),
            pltpu.SemaphoreType.DMA((2,)),
            pltpu.SemaphoreType.DMA,
        ],
    )(e, W1, b1, W2)


_BLK2 = 8192
_NB2 = (VOCAB + _BLK2 - 1) // _BLK2


def _k2_body(logits_ref, b2_ref, logz_ref, out_ref):
    out_ref[...] = logits_ref[...] + b2_ref[...] - logz_ref[0, 0]


def _tc_norm(logits, b2r, logz):
    return pl.pallas_call(
        _k2_body,
        grid=(_NB2,),
        in_specs=[
            pl.BlockSpec((1, _BLK2), lambda i: (0, i)),
            pl.BlockSpec((1, _BLK2), lambda i: (0, i)),
            pl.BlockSpec((1, D), lambda i: (0, 0)),
        ],
        out_specs=pl.BlockSpec((1, _BLK2), lambda i: (0, i)),
        out_shape=jax.ShapeDtypeStruct((1, VOCAB), jnp.float32),
    )(logits, b2r, logz)


def kernel(inputs, emb, W1, b1, W2, b2):
    idx2 = inputs.astype(jnp.int32).reshape(_CH, _CHN)
    e = _sc_gather_sum(idx2, emb).reshape(1, D)
    logits, logz = _tc_logits(e, W1, b1.reshape(1, D), W2)
    return _tc_norm(logits, b2.reshape(1, VOCAB), logz)


# speedup vs baseline: 1.4432x; 1.4432x over previous
import jax
import jax.numpy as jnp
from jax import lax
from jax.experimental import pallas as pl
from jax.experimental.pallas import tpu as pltpu

VOCAB = 100000
D = 128
CTX = 200

_BLK = 4096
_NB = (VOCAB + _BLK - 1) // _BLK
_LAST = VOCAB - (_NB - 1) * _BLK


def _k1_body(idx_ref, w1_ref, b1_ref, emb_hbm, w2_hbm, out_ref, logz_ref,
             h_ref, gbuf, buf, last_buf, m_arr, s_arr, gsem, sem, sem_last):
    i = pl.program_id(0)

    @pl.when(i == 0)
    def _():
        pltpu.make_async_copy(
            w2_hbm.at[pl.ds(0, _BLK), :], buf.at[0], sem.at[0]).start()
        for r in range(CTX):
            pltpu.make_async_copy(
                emb_hbm.at[pl.ds(idx_ref[r], 1), :],
                gbuf.at[pl.ds(r, 1), :], gsem).start()
        pltpu.make_async_copy(emb_hbm.at[pl.ds(0, CTX), :], gbuf, gsem).wait()
        e = jnp.sum(gbuf[...], axis=0, keepdims=True)
        h = jnp.dot(e, w1_ref[...].T,
                    preferred_element_type=jnp.float32) + b1_ref[...]
        h_ref[...] = jnp.maximum(h, 0.0)

    @pl.when(i + 1 < _NB - 1)
    def _():
        slot = lax.rem(i + 1, 2)
        pltpu.make_async_copy(
            w2_hbm.at[pl.ds((i + 1) * _BLK, _BLK), :], buf.at[slot],
            sem.at[slot]).start()

    @pl.when(i + 1 == _NB - 1)
    def _():
        pltpu.make_async_copy(
            w2_hbm.at[pl.ds((_NB - 1) * _BLK, _LAST), :], last_buf,
            sem_last).start()

    def stats(logits, bm):
        m_arr[i] = bm
        s_arr[i] = jnp.sum(jnp.exp(logits - bm))

    @pl.when(i < _NB - 1)
    def _():
        slot = lax.rem(i, 2)
        pltpu.make_async_copy(
            w2_hbm.at[pl.ds(i * _BLK, _BLK), :], buf.at[slot],
            sem.at[slot]).wait()
        logits = lax.dot_general(
            h_ref[...], buf[slot], (((1,), (1,)), ((), ())),
            preferred_element_type=jnp.float32)
        out_ref[...] = logits
        stats(logits, jnp.max(logits))

    @pl.when(i == _NB - 1)
    def _():
        pltpu.make_async_copy(
            w2_hbm.at[pl.ds((_NB - 1) * _BLK, _LAST), :], last_buf,
            sem_last).wait()
        logits = lax.dot_general(
            h_ref[...], last_buf[...], (((1,), (1,)), ((), ())),
            preferred_element_type=jnp.float32)
        out_ref[0:1, 0:_LAST] = logits
        stats(logits, jnp.max(logits))

        def comb(j, carry):
            m, s = carry
            mj = m_arr[j]
            mn = jnp.maximum(m, mj)
            return mn, s * jnp.exp(m - mn) + s_arr[j] * jnp.exp(mj - mn)

        m0, s0 = lax.fori_loop(1, _NB, comb, (m_arr[0], s_arr[0]))
        logz_ref[...] = jnp.full((1, D), m0 + jnp.log(s0), jnp.float32)


def _tc_main(idx, W1, b1, emb, W2):
    return pl.pallas_call(
        _k1_body,
        grid=(_NB,),
        in_specs=[
            pl.BlockSpec(memory_space=pltpu.SMEM),
            pl.BlockSpec((D, D), lambda i: (0, 0)),
            pl.BlockSpec((1, D), lambda i: (0, 0)),
            pl.BlockSpec(memory_space=pl.ANY),
            pl.BlockSpec(memory_space=pl.ANY),
        ],
        out_specs=[
            pl.BlockSpec((1, _BLK), lambda i: (0, i)),
            pl.BlockSpec((1, D), lambda i: (0, 0)),
        ],
        out_shape=[
            jax.ShapeDtypeStruct((1, VOCAB), jnp.float32),
            jax.ShapeDtypeStruct((1, D), jnp.float32),
        ],
        scratch_shapes=[
            pltpu.VMEM((1, D), jnp.float32),
            pltpu.VMEM((CTX, D), jnp.float32),
            pltpu.VMEM((2, _BLK, D), jnp.float32),
            pltpu.VMEM((_LAST, D), jnp.float32),
            pltpu.SMEM((_NB,), jnp.float32),
            pltpu.SMEM((_NB,), jnp.float32),
            pltpu.SemaphoreType.DMA,
            pltpu.SemaphoreType.DMA((2,)),
            pltpu.SemaphoreType.DMA,
        ],
    )(idx, W1, b1, emb, W2)


_BLK2 = 8192
_NB2 = (VOCAB + _BLK2 - 1) // _BLK2


def _k2_body(logits_ref, b2_ref, logz_ref, out_ref):
    out_ref[...] = logits_ref[...] + b2_ref[...] - logz_ref[0, 0]


def _tc_norm(logits, b2r, logz):
    return pl.pallas_call(
        _k2_body,
        grid=(_NB2,),
        in_specs=[
            pl.BlockSpec((1, _BLK2), lambda i: (0, i)),
            pl.BlockSpec((1, _BLK2), lambda i: (0, i)),
            pl.BlockSpec((1, D), lambda i: (0, 0)),
        ],
        out_specs=pl.BlockSpec((1, _BLK2), lambda i: (0, i)),
        out_shape=jax.ShapeDtypeStruct((1, VOCAB), jnp.float32),
    )(logits, b2r, logz)


def kernel(inputs, emb, W1, b1, W2, b2):
    idx = inputs.astype(jnp.int32)
    logits, logz = _tc_main(idx, W1, b1.reshape(1, D), emb, W2)
    return _tc_norm(logits, b2.reshape(1, VOCAB), logz)


# speedup vs baseline: 1.4944x; 1.0355x over previous
import jax
import jax.numpy as jnp
from jax import lax
from jax.experimental import pallas as pl
from jax.experimental.pallas import tpu as pltpu

VOCAB = 100000
D = 128
CTX = 200

_BLK = 4096
_NA = (VOCAB + _BLK - 1) // _BLK
_LAST = VOCAB - (_NA - 1) * _BLK
_BLK2 = 8192
_NB = (VOCAB + _BLK2 - 1) // _BLK2
_LBUF = _NB * _BLK2


def _body(idx_ref, w1_ref, b1_ref, b2_ref, emb_hbm, w2_hbm, out_ref,
          h_ref, gbuf, buf, last_buf, lbuf, m_arr, s_arr, logz_s,
          gsem, sem, sem_last):
    i = pl.program_id(0)

    @pl.when(i == 0)
    def _():
        pltpu.make_async_copy(
            w2_hbm.at[pl.ds(0, _BLK), :], buf.at[0], sem.at[0]).start()
        for r in range(CTX):
            pltpu.make_async_copy(
                emb_hbm.at[pl.ds(idx_ref[r], 1), :],
                gbuf.at[pl.ds(r, 1), :], gsem).start()
        pltpu.make_async_copy(emb_hbm.at[pl.ds(0, CTX), :], gbuf, gsem).wait()
        e = jnp.sum(gbuf[...], axis=0, keepdims=True)
        h = jnp.dot(e, w1_ref[...].T,
                    preferred_element_type=jnp.float32) + b1_ref[...]
        h_ref[...] = jnp.maximum(h, 0.0).astype(jnp.bfloat16)

    @pl.when(i + 1 < _NA - 1)
    def _():
        slot = lax.rem(i + 1, 2)
        pltpu.make_async_copy(
            w2_hbm.at[pl.ds((i + 1) * _BLK, _BLK), :], buf.at[slot],
            sem.at[slot]).start()

    @pl.when(i + 1 == _NA - 1)
    def _():
        pltpu.make_async_copy(
            w2_hbm.at[pl.ds((_NA - 1) * _BLK, _LAST), :], last_buf,
            sem_last).start()

    def stats(k, logits):
        bm = jnp.max(logits)
        m_arr[k] = bm
        s_arr[k] = jnp.sum(jnp.exp(logits - bm))

    @pl.when(i < _NA - 1)
    def _():
        slot = lax.rem(i, 2)
        pltpu.make_async_copy(
            w2_hbm.at[pl.ds(i * _BLK, _BLK), :], buf.at[slot],
            sem.at[slot]).wait()
        logits = lax.dot_general(
            h_ref[...], buf[slot].astype(jnp.bfloat16),
            (((1,), (1,)), ((), ())), preferred_element_type=jnp.float32)
        lbuf[0:1, pl.ds(i * _BLK, _BLK)] = logits
        stats(i, logits)

    @pl.when(i == _NA - 1)
    def _():
        pltpu.make_async_copy(
            w2_hbm.at[pl.ds((_NA - 1) * _BLK, _LAST), :], last_buf,
            sem_last).wait()
        logits = lax.dot_general(
            h_ref[...], last_buf[...].astype(jnp.bfloat16),
            (((1,), (1,)), ((), ())), preferred_element_type=jnp.float32)
        lbuf[0:1, pl.ds((_NA - 1) * _BLK, _LAST)] = logits
        stats(_NA - 1, logits)

    @pl.when(i == _NA)
    def _():
        def comb(j, carry):
            m, s = carry
            mj = m_arr[j]
            mn = jnp.maximum(m, mj)
            return mn, s * jnp.exp(m - mn) + s_arr[j] * jnp.exp(mj - mn)

        m0, s0 = lax.fori_loop(1, _NA, comb, (m_arr[0], s_arr[0]))
        logz_s[0] = m0 + jnp.log(s0)

    @pl.when(i >= _NA)
    def _():
        j = i - _NA
        out_ref[...] = (lbuf[0:1, pl.ds(j * _BLK2, _BLK2)]
                        + b2_ref[...] - logz_s[0])


def _tc_main(idx, W1, b1, b2r, emb, W2):
    def b2_map(i):
        return (0, jnp.maximum(i - _NA, 0))

    return pl.pallas_call(
        _body,
        grid=(_NA + _NB,),
        in_specs=[
            pl.BlockSpec(memory_space=pltpu.SMEM),
            pl.BlockSpec((D, D), lambda i: (0, 0)),
            pl.BlockSpec((1, D), lambda i: (0, 0)),
            pl.BlockSpec((1, _BLK2), b2_map),
            pl.BlockSpec(memory_space=pl.ANY),
            pl.BlockSpec(memory_space=pl.ANY),
        ],
        out_specs=pl.BlockSpec((1, _BLK2), b2_map),
        out_shape=jax.ShapeDtypeStruct((1, VOCAB), jnp.float32),
        scratch_shapes=[
            pltpu.VMEM((1, D), jnp.bfloat16),
            pltpu.VMEM((CTX, D), jnp.float32),
            pltpu.VMEM((2, _BLK, D), jnp.float32),
            pltpu.VMEM((_LAST, D), jnp.float32),
            pltpu.VMEM((1, _LBUF), jnp.float32),
            pltpu.SMEM((_NA,), jnp.float32),
            pltpu.SMEM((_NA,), jnp.float32),
            pltpu.SMEM((1,), jnp.float32),
            pltpu.SemaphoreType.DMA,
            pltpu.SemaphoreType.DMA((2,)),
            pltpu.SemaphoreType.DMA,
        ],
    )(idx, W1, b1, b2r, emb, W2)


def kernel(inputs, emb, W1, b1, W2, b2):
    idx = inputs.astype(jnp.int32)
    return _tc_main(idx, W1, b1.reshape(1, D), b2.reshape(1, VOCAB), emb, W2)


# speedup vs baseline: 1.8703x; 1.2515x over previous
import jax
import jax.numpy as jnp
from jax import lax
from jax.experimental import pallas as pl
from jax.experimental.pallas import tpu as pltpu

VOCAB = 100000
D = 128
CTX = 200

_BLK = 8192
_NA = (VOCAB + _BLK - 1) // _BLK
_LAST = VOCAB - (_NA - 1) * _BLK
_BLK2 = 8192
_NB = (VOCAB + _BLK2 - 1) // _BLK2
_LBUF = _NB * _BLK2


def _body(idx_ref, w1_ref, b1_ref, b2_ref, emb_hbm, w2_hbm, out_ref,
          h_ref, gbuf, buf, last_buf, lbuf, m_arr, s_arr, logz_s,
          gsem, sem, sem_last):
    i = pl.program_id(0)

    @pl.when(i == 0)
    def _():
        pltpu.make_async_copy(
            w2_hbm.at[pl.ds(0, _BLK), :], buf.at[0], sem.at[0]).start()
        for r in range(CTX):
            pltpu.make_async_copy(
                emb_hbm.at[pl.ds(idx_ref[r], 1), :],
                gbuf.at[pl.ds(r, 1), :], gsem).start()
        pltpu.make_async_copy(emb_hbm.at[pl.ds(0, CTX), :], gbuf, gsem).wait()
        e = jnp.sum(gbuf[...], axis=0, keepdims=True)
        h = jnp.dot(e, w1_ref[...].T,
                    preferred_element_type=jnp.float32) + b1_ref[...]
        h_ref[...] = jnp.maximum(h, 0.0).astype(jnp.bfloat16)

    @pl.when(i + 1 < _NA - 1)
    def _():
        slot = lax.rem(i + 1, 2)
        pltpu.make_async_copy(
            w2_hbm.at[pl.ds((i + 1) * _BLK, _BLK), :], buf.at[slot],
            sem.at[slot]).start()

    @pl.when(i + 1 == _NA - 1)
    def _():
        pltpu.make_async_copy(
            w2_hbm.at[pl.ds((_NA - 1) * _BLK, _LAST), :], last_buf,
            sem_last).start()

    def stats(k, logits):
        bm = jnp.max(logits)
        m_arr[k] = bm
        s_arr[k] = jnp.sum(jnp.exp(logits - bm))

    @pl.when(i < _NA - 1)
    def _():
        slot = lax.rem(i, 2)
        pltpu.make_async_copy(
            w2_hbm.at[pl.ds(i * _BLK, _BLK), :], buf.at[slot],
            sem.at[slot]).wait()
        logits = lax.dot_general(
            h_ref[...], buf[slot].astype(jnp.bfloat16),
            (((1,), (1,)), ((), ())), preferred_element_type=jnp.float32)
        lbuf[0:1, pl.ds(i * _BLK, _BLK)] = logits
        stats(i, logits)

    @pl.when(i == _NA - 1)
    def _():
        pltpu.make_async_copy(
            w2_hbm.at[pl.ds((_NA - 1) * _BLK, _LAST), :], last_buf,
            sem_last).wait()
        logits = lax.dot_general(
            h_ref[...], last_buf[...].astype(jnp.bfloat16),
            (((1,), (1,)), ((), ())), preferred_element_type=jnp.float32)
        lbuf[0:1, pl.ds((_NA - 1) * _BLK, _LAST)] = logits
        stats(_NA - 1, logits)

    @pl.when(i == _NA)
    def _():
        def comb(j, carry):
            m, s = carry
            mj = m_arr[j]
            mn = jnp.maximum(m, mj)
            return mn, s * jnp.exp(m - mn) + s_arr[j] * jnp.exp(mj - mn)

        m0, s0 = lax.fori_loop(1, _NA, comb, (m_arr[0], s_arr[0]))
        logz_s[0] = m0 + jnp.log(s0)

    @pl.when(i >= _NA)
    def _():
        j = i - _NA
        out_ref[...] = (lbuf[0:1, pl.ds(j * _BLK2, _BLK2)]
                        + b2_ref[...] - logz_s[0])


def _tc_main(idx, W1, b1, b2r, emb, W2):
    def b2_map(i):
        return (0, jnp.maximum(i - _NA, 0))

    return pl.pallas_call(
        _body,
        grid=(_NA + _NB,),
        in_specs=[
            pl.BlockSpec(memory_space=pltpu.SMEM),
            pl.BlockSpec((D, D), lambda i: (0, 0)),
            pl.BlockSpec((1, D), lambda i: (0, 0)),
            pl.BlockSpec((1, _BLK2), b2_map),
            pl.BlockSpec(memory_space=pl.ANY),
            pl.BlockSpec(memory_space=pl.ANY),
        ],
        out_specs=pl.BlockSpec((1, _BLK2), b2_map),
        out_shape=jax.ShapeDtypeStruct((1, VOCAB), jnp.float32),
        scratch_shapes=[
            pltpu.VMEM((1, D), jnp.bfloat16),
            pltpu.VMEM((CTX, D), jnp.float32),
            pltpu.VMEM((2, _BLK, D), jnp.float32),
            pltpu.VMEM((_LAST, D), jnp.float32),
            pltpu.VMEM((1, _LBUF), jnp.float32),
            pltpu.SMEM((_NA,), jnp.float32),
            pltpu.SMEM((_NA,), jnp.float32),
            pltpu.SMEM((1,), jnp.float32),
            pltpu.SemaphoreType.DMA,
            pltpu.SemaphoreType.DMA((2,)),
            pltpu.SemaphoreType.DMA,
        ],
    )(idx, W1, b1, b2r, emb, W2)


def kernel(inputs, emb, W1, b1, W2, b2):
    idx = inputs.astype(jnp.int32)
    return _tc_main(idx, W1, b1.reshape(1, D), b2.reshape(1, VOCAB), emb, W2)


# speedup vs baseline: 2.1074x; 1.1268x over previous
import jax
import jax.numpy as jnp
from jax import lax
from jax.experimental import pallas as pl
from jax.experimental.pallas import tpu as pltpu

VOCAB = 100000
D = 128
CTX = 200

_BLK = 16384
_NA = (VOCAB + _BLK - 1) // _BLK
_LAST = VOCAB - (_NA - 1) * _BLK
_BLK2 = 8192
_NB = (VOCAB + _BLK2 - 1) // _BLK2
_LBUF = _NB * _BLK2


def _body(idx_ref, w1_ref, b1_ref, b2_ref, emb_hbm, w2_hbm, out_ref,
          h_ref, gbuf, buf, last_buf, lbuf, m_arr, s_arr, logz_s,
          gsem, sem, sem_last):
    i = pl.program_id(0)

    @pl.when(i == 0)
    def _():
        pltpu.make_async_copy(
            w2_hbm.at[pl.ds(0, _BLK), :], buf.at[0], sem.at[0]).start()
        for r in range(CTX):
            pltpu.make_async_copy(
                emb_hbm.at[pl.ds(idx_ref[r], 1), :],
                gbuf.at[pl.ds(r, 1), :], gsem).start()
        pltpu.make_async_copy(emb_hbm.at[pl.ds(0, CTX), :], gbuf, gsem).wait()
        e = jnp.sum(gbuf[...], axis=0, keepdims=True)
        h = jnp.dot(e, w1_ref[...].T,
                    preferred_element_type=jnp.float32) + b1_ref[...]
        h_ref[...] = jnp.maximum(h, 0.0).astype(jnp.bfloat16)

    @pl.when(i + 1 < _NA - 1)
    def _():
        slot = lax.rem(i + 1, 2)
        pltpu.make_async_copy(
            w2_hbm.at[pl.ds((i + 1) * _BLK, _BLK), :], buf.at[slot],
            sem.at[slot]).start()

    @pl.when(i + 1 == _NA - 1)
    def _():
        pltpu.make_async_copy(
            w2_hbm.at[pl.ds((_NA - 1) * _BLK, _LAST), :], last_buf,
            sem_last).start()

    def stats(k, logits):
        bm = jnp.max(logits)
        m_arr[k] = bm
        s_arr[k] = jnp.sum(jnp.exp(logits - bm))

    @pl.when(i < _NA - 1)
    def _():
        slot = lax.rem(i, 2)
        pltpu.make_async_copy(
            w2_hbm.at[pl.ds(i * _BLK, _BLK), :], buf.at[slot],
            sem.at[slot]).wait()
        logits = lax.dot_general(
            h_ref[...], buf[slot].astype(jnp.bfloat16),
            (((1,), (1,)), ((), ())), preferred_element_type=jnp.float32)
        lbuf[0:1, pl.ds(i * _BLK, _BLK)] = logits
        stats(i, logits)

    @pl.when(i == _NA - 1)
    def _():
        pltpu.make_async_copy(
            w2_hbm.at[pl.ds((_NA - 1) * _BLK, _LAST), :], last_buf,
            sem_last).wait()
        logits = lax.dot_general(
            h_ref[...], last_buf[...].astype(jnp.bfloat16),
            (((1,), (1,)), ((), ())), preferred_element_type=jnp.float32)
        lbuf[0:1, pl.ds((_NA - 1) * _BLK, _LAST)] = logits
        stats(_NA - 1, logits)

    @pl.when(i == _NA)
    def _():
        def comb(j, carry):
            m, s = carry
            mj = m_arr[j]
            mn = jnp.maximum(m, mj)
            return mn, s * jnp.exp(m - mn) + s_arr[j] * jnp.exp(mj - mn)

        m0, s0 = lax.fori_loop(1, _NA, comb, (m_arr[0], s_arr[0]))
        logz_s[0] = m0 + jnp.log(s0)

    @pl.when(i >= _NA)
    def _():
        j = i - _NA
        out_ref[...] = (lbuf[0:1, pl.ds(j * _BLK2, _BLK2)]
                        + b2_ref[...] - logz_s[0])


def _tc_main(idx, W1, b1, b2r, emb, W2):
    def b2_map(i):
        return (0, jnp.maximum(i - _NA, 0))

    return pl.pallas_call(
        _body,
        grid=(_NA + _NB,),
        in_specs=[
            pl.BlockSpec(memory_space=pltpu.SMEM),
            pl.BlockSpec((D, D), lambda i: (0, 0)),
            pl.BlockSpec((1, D), lambda i: (0, 0)),
            pl.BlockSpec((1, _BLK2), b2_map),
            pl.BlockSpec(memory_space=pl.ANY),
            pl.BlockSpec(memory_space=pl.ANY),
        ],
        out_specs=pl.BlockSpec((1, _BLK2), b2_map),
        out_shape=jax.ShapeDtypeStruct((1, VOCAB), jnp.float32),
        scratch_shapes=[
            pltpu.VMEM((1, D), jnp.bfloat16),
            pltpu.VMEM((CTX, D), jnp.float32),
            pltpu.VMEM((2, _BLK, D), jnp.float32),
            pltpu.VMEM((_LAST, D), jnp.float32),
            pltpu.VMEM((1, _LBUF), jnp.float32),
            pltpu.SMEM((_NA,), jnp.float32),
            pltpu.SMEM((_NA,), jnp.float32),
            pltpu.SMEM((1,), jnp.float32),
            pltpu.SemaphoreType.DMA,
            pltpu.SemaphoreType.DMA((2,)),
            pltpu.SemaphoreType.DMA,
        ],
    )(idx, W1, b1, b2r, emb, W2)


def kernel(inputs, emb, W1, b1, W2, b2):
    idx = inputs.astype(jnp.int32)
    return _tc_main(idx, W1, b1.reshape(1, D), b2.reshape(1, VOCAB), emb, W2)
